# Initial kernel scaffold; baseline (speedup 1.0000x reference)
import functools

import jax
import jax.numpy as jnp
from jax import lax
from jax.experimental import pallas as pl
from jax.experimental.pallas import tpu as pltpu
from jax.experimental.pallas import tpu_sc as plsc

NC = 2
NS = 16
NW = NC * NS
CH = 80
WIDE = 144
BE = 512
BN = 1000


def _silu(v):
    return v * jax.nn.sigmoid(v)


def _mesh():
    return plsc.VectorSubcoreMesh(core_axis_name="c", subcore_axis_name="s")


@functools.cache
def _sc_gather(E, W):
    EPW = E // NW
    assert EPW % CH == 0

    @functools.partial(
        pl.kernel,
        out_type=jax.ShapeDtypeStruct((E, W), jnp.float32),
        mesh=_mesh(),
        scratch_types=[
            pltpu.VMEM((CH,), jnp.int32),
            pltpu.VMEM((CH,), jnp.int32),
            pltpu.VMEM((CH, W), jnp.float32),
            pltpu.SemaphoreType.DMA,
        ],
    )
    def k(td, ts, dsti, srci, g, idx_d, idx_s, buf, sem):
        wid = lax.axis_index("s") * NC + lax.axis_index("c")
        base = wid * EPW

        def body(i, carry):
            off = base + i * CH
            pltpu.sync_copy(dsti.at[pl.ds(off, CH)], idx_d)
            pltpu.sync_copy(srci.at[pl.ds(off, CH)], idx_s)
            pltpu.async_copy(td.at[idx_d], buf, sem).wait()
            pltpu.async_copy(ts.at[idx_s], buf, sem, add=True).wait()
            pltpu.sync_copy(buf, g.at[pl.ds(off, CH)])
            return carry

        lax.fori_loop(0, EPW // CH, body, 0)

    return k


@functools.cache
def _sc_scatter(E, N, W):
    EPW = E // NW
    NPT = N // NS
    assert EPW % CH == 0 and N % NS == 0

    @functools.partial(
        pl.kernel,
        out_type=jax.ShapeDtypeStruct((NC * N, W), jnp.float32),
        mesh=_mesh(),
        scratch_types=[
            pltpu.VMEM((CH,), jnp.int32),
            pltpu.VMEM((CH, W), jnp.float32),
            pltpu.VMEM_SHARED((N, W), jnp.float32),
            pltpu.SemaphoreType.DMA,
        ],
    )
    def k(m, dsti, zrows, parts, idx, buf, acc, sem):
        cid = lax.axis_index("c")
        sid = lax.axis_index("s")
        wid = sid * NC + cid
        base = wid * EPW
        pltpu.sync_copy(zrows.at[pl.ds(sid * NPT, NPT)], acc.at[pl.ds(sid * NPT, NPT)])
        plsc.subcore_barrier()

        def body(i, carry):
            off = base + i * CH
            pltpu.sync_copy(dsti.at[pl.ds(off, CH)], idx)
            pltpu.sync_copy(m.at[pl.ds(off, CH)], buf)
            pltpu.sync_copy(buf, acc.at[idx], add=True)
            return carry

        lax.fori_loop(0, EPW // CH, body, 0)
        plsc.subcore_barrier()
        pltpu.sync_copy(acc.at[pl.ds(sid * NPT, NPT)],
                        parts.at[pl.ds(cid * N + sid * NPT, NPT)])

    return k


def _const_spec(shape):
    return pl.BlockSpec(shape, lambda i: (0,) * len(shape))


@functools.cache
def _tc_tables(N, D):
    def body(h_ref, wa_ref, wb_ref, a_ref, b_ref):
        h = h_ref[...]
        a_ref[...] = jnp.dot(h, wa_ref[...], preferred_element_type=jnp.float32)
        b_ref[...] = jnp.dot(h, wb_ref[...], preferred_element_type=jnp.float32)

    return pl.pallas_call(
        body,
        grid=(N // BN,),
        in_specs=[
            pl.BlockSpec((BN, D), lambda i: (i, 0)),
            _const_spec((D, D)),
            _const_spec((D, D)),
        ],
        out_specs=[pl.BlockSpec((BN, D), lambda i: (i, 0))] * 2,
        out_shape=[jax.ShapeDtypeStruct((N, D), jnp.float32)] * 2,
    )


@functools.cache
def _tc_edge(E, D, ED, lean):
    OW = 16 if lean else WIDE

    def body(g_ref, ea_ref, w1ea_ref, wd2_ref, be1_ref, we2_ref, be2_ref,
             wc1_ref, bc1_ref, wc2_ref, bc2_ref, out_ref):
        g = g_ref[...]
        s = g[:, :D]
        rel = g[:, D:D + 4]
        d2 = jnp.sum(rel * rel, axis=1, keepdims=True)
        pre = (s + d2 * wd2_ref[...] + be1_ref[...]
               + jnp.dot(ea_ref[...], w1ea_ref[...], preferred_element_type=jnp.float32))
        m1 = _silu(pre)
        m = _silu(jnp.dot(m1, we2_ref[...], preferred_element_type=jnp.float32)
                  + be2_ref[...])
        t = _silu(jnp.dot(m, wc1_ref[...], preferred_element_type=jnp.float32)
                  + bc1_ref[...])
        c = jnp.sum(t * wc2_ref[...], axis=1, keepdims=True) + bc2_ref[...]
        tail = jnp.concatenate(
            [rel * c, jnp.ones((BE, 1), jnp.float32), jnp.zeros((BE, 11), jnp.float32)],
            axis=1)
        if lean:
            out_ref[...] = tail
        else:
            out_ref[:, :D] = m
            out_ref[:, D:D + 16] = tail

    return pl.pallas_call(
        body,
        grid=(E // BE,),
        in_specs=[
            pl.BlockSpec((BE, WIDE), lambda i: (i, 0)),
            pl.BlockSpec((BE, ED), lambda i: (i, 0)),
            _const_spec((ED, D)),
            _const_spec((1, D)),
            _const_spec((1, D)),
            _const_spec((D, D)),
            _const_spec((1, D)),
            _const_spec((D, D)),
            _const_spec((1, D)),
            _const_spec((1, D)),
            _const_spec((1, 1)),
        ],
        out_specs=pl.BlockSpec((BE, OW), lambda i: (i, 0)),
        out_shape=jax.ShapeDtypeStruct((E, OW), jnp.float32),
    )


@functools.cache
def _tc_node(N, D, first):
    def body(*refs):
        if first:
            (parts_ref, h_ref, pos_ref, wn1a_ref, wn1b_ref, bn1_ref, wn2_ref,
             bn2_ref, wa_ref, wb_ref,
             h_out, pos_out, a_out, b_out, counts_out) = refs
        else:
            (parts_ref, h_ref, pos_ref, counts_ref, wn1a_ref, wn1b_ref, bn1_ref,
             wn2_ref, bn2_ref, wa_ref, wb_ref,
             h_out, pos_out, a_out, b_out) = refs
        p = parts_ref[...]
        ps = p[0] + p[1]
        agg = ps[:, :D]
        pd = ps[:, D:D + 4]
        if first:
            counts = ps[:, D + 4:D + 5]
            counts_out[...] = counts
        else:
            counts = counts_ref[...]
        denom = jnp.maximum(counts, 1.0)
        pos_out[...] = pos_ref[...] + pd / denom
        h = h_ref[...]
        u1 = _silu(jnp.dot(h, wn1a_ref[...], preferred_element_type=jnp.float32)
                   + jnp.dot(agg, wn1b_ref[...], preferred_element_type=jnp.float32)
                   + bn1_ref[...])
        ho = h + jnp.dot(u1, wn2_ref[...], preferred_element_type=jnp.float32) + bn2_ref[...]
        h_out[...] = ho
        a_out[...] = jnp.dot(ho, wa_ref[...], preferred_element_type=jnp.float32)
        b_out[...] = jnp.dot(ho, wb_ref[...], preferred_element_type=jnp.float32)

    in_specs = [
        pl.BlockSpec((NC, BN, WIDE), lambda i: (0, i, 0)),
        pl.BlockSpec((BN, D), lambda i: (i, 0)),
        pl.BlockSpec((BN, 4), lambda i: (i, 0)),
    ]
    if not first:
        in_specs.append(pl.BlockSpec((BN, 1), lambda i: (i, 0)))
    in_specs += [
        _const_spec((D, D)), _const_spec((D, D)), _const_spec((1, D)),
        _const_spec((D, D)), _const_spec((1, D)),
        _const_spec((D, D)), _const_spec((D, D)),
    ]
    out_specs = [
        pl.BlockSpec((BN, D), lambda i: (i, 0)),
        pl.BlockSpec((BN, 4), lambda i: (i, 0)),
        pl.BlockSpec((BN, D), lambda i: (i, 0)),
        pl.BlockSpec((BN, D), lambda i: (i, 0)),
    ]
    out_shape = [
        jax.ShapeDtypeStruct((N, D), jnp.float32),
        jax.ShapeDtypeStruct((N, 4), jnp.float32),
        jax.ShapeDtypeStruct((N, D), jnp.float32),
        jax.ShapeDtypeStruct((N, D), jnp.float32),
    ]
    if first:
        out_specs.append(pl.BlockSpec((BN, 1), lambda i: (i, 0)))
        out_shape.append(jax.ShapeDtypeStruct((N, 1), jnp.float32))
    return pl.pallas_call(
        body,
        grid=(N // BN,),
        in_specs=in_specs,
        out_specs=out_specs,
        out_shape=out_shape,
    )


@functools.cache
def _tc_pos(N):
    def body(parts_ref, pos_ref, counts_ref, pos_out):
        p = parts_ref[...]
        ps = p[0] + p[1]
        pd = ps[:, :4]
        denom = jnp.maximum(counts_ref[...], 1.0)
        pos_out[...] = pos_ref[...] + pd / denom

    return pl.pallas_call(
        body,
        grid=(N // BN,),
        in_specs=[
            pl.BlockSpec((NC, BN, 16), lambda i: (0, i, 0)),
            pl.BlockSpec((BN, 4), lambda i: (i, 0)),
            pl.BlockSpec((BN, 1), lambda i: (i, 0)),
        ],
        out_specs=pl.BlockSpec((BN, 4), lambda i: (i, 0)),
        out_shape=jax.ShapeDtypeStruct((N, 4), jnp.float32),
    )


def kernel(x, pos, edge_index, edge_attr, We1, be1, We2, be2,
           Wc1, bc1, Wc2, bc2, Wn1, bn1, Wn2, bn2):
    N, D = x.shape
    E = edge_index.shape[1]
    ED = edge_attr.shape[1]
    L = We1.shape[0]
    src = edge_index[0]
    dst = edge_index[1]

    pos4 = jnp.pad(pos, ((0, 0), (0, 1)))
    zpad = jnp.zeros((N, WIDE - D - 4), jnp.float32)
    zwide = jnp.zeros((N, WIDE), jnp.float32)
    z16 = jnp.zeros((N, 16), jnp.float32)

    W1a = We1[:, :D, :]
    W1b = We1[:, D:2 * D, :]
    wd2 = We1[:, 2 * D:2 * D + 1, :]
    W1ea = We1[:, 2 * D + 1:, :]
    Wn1a = Wn1[:, :D, :]
    Wn1b = Wn1[:, D:, :]
    wc2row = jnp.transpose(Wc2, (0, 2, 1))

    h = x
    A, B = _tc_tables(N, D)(x, W1a[0], W1b[0])
    counts = None
    for l in range(L):
        td = jnp.concatenate([A, pos4, zpad], axis=1)
        ts = jnp.concatenate([B, -pos4, zpad], axis=1)
        g = _sc_gather(E, WIDE)(td, ts, dst, src)
        last = l == L - 1
        m = _tc_edge(E, D, ED, last)(
            g, edge_attr, W1ea[l], wd2[l], be1[l][None], We2[l], be2[l][None],
            Wc1[l], bc1[l][None], wc2row[l], bc2[l][None])
        if not last:
            parts = _sc_scatter(E, N, WIDE)(m, dst, zwide).reshape(NC, N, WIDE)
            if l == 0:
                h, pos4, A, B, counts = _tc_node(N, D, True)(
                    parts, h, pos4, Wn1a[l], Wn1b[l], bn1[l][None], Wn2[l],
                    bn2[l][None], W1a[l + 1], W1b[l + 1])
            else:
                h, pos4, A, B = _tc_node(N, D, False)(
                    parts, h, pos4, counts, Wn1a[l], Wn1b[l], bn1[l][None],
                    Wn2[l], bn2[l][None], W1a[l + 1], W1b[l + 1])
        else:
            parts = _sc_scatter(E, N, 16)(m, dst, z16).reshape(NC, N, 16)
            pos4 = _tc_pos(N)(parts, pos4, counts)
    return pos4[:, :3]


# speedup vs baseline: 2.3047x; 2.3047x over previous
import functools

import jax
import jax.numpy as jnp
from jax import lax
from jax.experimental import pallas as pl
from jax.experimental.pallas import tpu as pltpu
from jax.experimental.pallas import tpu_sc as plsc

NC = 2
NS = 16
NW = NC * NS
LANES = 16
CH = 80
SW = 8
BE = 512
BN = 1000


def _silu(v):
    return v * jax.nn.sigmoid(v)


def _mesh():
    return plsc.VectorSubcoreMesh(core_axis_name="c", subcore_axis_name="s")


def _iota16():
    return jnp.arange(LANES, dtype=jnp.int32)


@functools.cache
def _sc_gather(E, N, D):
    EPW = E // NW
    assert EPW % CH == 0

    @functools.partial(
        pl.kernel,
        out_type=(
            jax.ShapeDtypeStruct((E, D), jnp.float32),
            jax.ShapeDtypeStruct((E * SW,), jnp.float32),
        ),
        mesh=_mesh(),
        compiler_params=pltpu.CompilerParams(needs_layout_passes=False),
        scratch_types=[
            pltpu.VMEM((CH,), jnp.int32),
            pltpu.VMEM((CH,), jnp.int32),
            pltpu.VMEM((CH, D), jnp.float32),
            pltpu.VMEM((CH * SW,), jnp.float32),
            pltpu.VMEM((4 * N,), jnp.float32),
            pltpu.SemaphoreType.DMA,
        ],
    )
    def k(ta, tb, dsti, srci, posf, s_out, g2_out,
          idx_d, idx_s, buf, buf2, posv, sem):
        wid = lax.axis_index("s") * NC + lax.axis_index("c")
        base = wid * EPW
        pltpu.sync_copy(posf, posv)

        def body(i, carry):
            off = base + i * CH
            pltpu.sync_copy(dsti.at[pl.ds(off, CH)], idx_d)
            pltpu.sync_copy(srci.at[pl.ds(off, CH)], idx_s)
            cp1 = pltpu.async_copy(ta.at[idx_d], buf, sem)
            for j in range(CH // LANES):
                dm = idx_d[pl.ds(j * LANES, LANES)]
                sm = idx_s[pl.ds(j * LANES, LANES)]
                li = (j * LANES + _iota16()) * SW
                d2 = jnp.zeros((LANES,), jnp.float32)
                for c in range(3):
                    pd = plsc.load_gather(posv, [dm * 4 + c])
                    ps = plsc.load_gather(posv, [sm * 4 + c])
                    rel = pd - ps
                    d2 = d2 + rel * rel
                    plsc.store_scatter(buf2, [li + c], rel)
                plsc.store_scatter(buf2, [li + 3], d2)
            cp1.wait()
            pltpu.async_copy(tb.at[idx_s], buf, sem, add=True).wait()
            pltpu.sync_copy(buf, s_out.at[pl.ds(off, CH)])
            pltpu.sync_copy(buf2, g2_out.at[pl.ds(off * SW, CH * SW)])
            return carry

        lax.fori_loop(0, EPW // CH, body, 0)

    return k


@functools.cache
def _sc_scatter_wide(E, N, D):
    EPW = E // NW
    NP = -(-N // (NS * 8)) * (NS * 8)
    NPT = NP // NS
    assert EPW % CH == 0

    @functools.partial(
        pl.kernel,
        out_type=jax.ShapeDtypeStruct((NC * NP, D), jnp.float32),
        mesh=_mesh(),
        compiler_params=pltpu.CompilerParams(needs_layout_passes=False),
        scratch_types=[
            pltpu.VMEM((CH,), jnp.int32),
            pltpu.VMEM((CH, D), jnp.float32),
            pltpu.VMEM_SHARED((NP, D), jnp.float32),
            pltpu.SemaphoreType.DMA,
        ],
    )
    def k(m, dsti, zrows, partsm, idx, buf, accm, sem):
        cid = lax.axis_index("c")
        sid = lax.axis_index("s")
        wid = sid * NC + cid
        base = wid * EPW
        pltpu.sync_copy(zrows.at[pl.ds(sid * NPT, NPT)],
                        accm.at[pl.ds(sid * NPT, NPT)])
        plsc.subcore_barrier()

        def body(i, carry):
            off = base + i * CH
            pltpu.sync_copy(dsti.at[pl.ds(off, CH)], idx)
            pltpu.sync_copy(m.at[pl.ds(off, CH)], buf)
            pltpu.sync_copy(buf, accm.at[idx], add=True)
            return carry

        lax.fori_loop(0, EPW // CH, body, 0)
        plsc.subcore_barrier()
        pltpu.sync_copy(accm.at[pl.ds(sid * NPT, NPT)],
                        partsm.at[pl.ds(cid * NP + sid * NPT, NPT)])

    return k


@functools.cache
def _sc_scatter_narrow(E, N):
    EPW = E // NW
    assert EPW % CH == 0

    @functools.partial(
        pl.kernel,
        out_type=jax.ShapeDtypeStruct((NW * N * SW,), jnp.float32),
        mesh=_mesh(),
        compiler_params=pltpu.CompilerParams(needs_layout_passes=False),
        scratch_types=[
            pltpu.VMEM((CH,), jnp.int32),
            pltpu.VMEM((CH * SW,), jnp.float32),
            pltpu.VMEM((N * SW,), jnp.float32),
            pltpu.SemaphoreType.DMA,
        ],
    )
    def k(w2f, dsti, z8, parts32, idx, buf2, acc2, sem):
        wid = lax.axis_index("s") * NC + lax.axis_index("c")
        base = wid * EPW
        pltpu.sync_copy(z8, acc2)

        def body(i, carry):
            off = base + i * CH
            pltpu.sync_copy(dsti.at[pl.ds(off, CH)], idx)
            pltpu.sync_copy(w2f.at[pl.ds(off * SW, CH * SW)], buf2)
            for j in range(CH // LANES):
                dm = idx[pl.ds(j * LANES, LANES)]
                li = (j * LANES + _iota16()) * SW
                for c in range(4):
                    v = plsc.load_gather(buf2, [li + c])
                    plsc.addupdate_scatter(acc2, [dm * SW + c], v)
            return carry

        lax.fori_loop(0, EPW // CH, body, 0)
        pltpu.sync_copy(acc2, parts32.at[pl.ds(wid * N * SW, N * SW)])

    return k


def _const_spec(shape):
    return pl.BlockSpec(shape, lambda i: (0,) * len(shape))


@functools.cache
def _tc_tables(N, D):
    def body(h_ref, wa_ref, wb_ref, a_ref, b_ref):
        h = h_ref[...]
        a_ref[...] = jnp.dot(h, wa_ref[...], preferred_element_type=jnp.float32)
        b_ref[...] = jnp.dot(h, wb_ref[...], preferred_element_type=jnp.float32)

    return pl.pallas_call(
        body,
        grid=(N // BN,),
        in_specs=[
            pl.BlockSpec((BN, D), lambda i: (i, 0)),
            _const_spec((D, D)),
            _const_spec((D, D)),
        ],
        out_specs=[pl.BlockSpec((BN, D), lambda i: (i, 0))] * 2,
        out_shape=[jax.ShapeDtypeStruct((N, D), jnp.float32)] * 2,
    )


@functools.cache
def _tc_edge(E, D, ED, lean):
    def body(s_ref, g2_ref, ea_ref, w1ea_ref, wd2_ref, be1_ref, we2_ref,
             be2_ref, wc1_ref, bc1_ref, wc2_ref, bc2_ref, *outs):
        s = s_ref[...]
        g2 = g2_ref[...]
        d2 = g2[:, 3:4]
        pre = (s + d2 * wd2_ref[...] + be1_ref[...]
               + jnp.dot(ea_ref[...], w1ea_ref[...], preferred_element_type=jnp.float32))
        m1 = _silu(pre)
        m = _silu(jnp.dot(m1, we2_ref[...], preferred_element_type=jnp.float32)
                  + be2_ref[...])
        t = _silu(jnp.dot(m, wc1_ref[...], preferred_element_type=jnp.float32)
                  + bc1_ref[...])
        c = jnp.sum(t * wc2_ref[...], axis=1, keepdims=True) + bc2_ref[...]
        w2 = jnp.concatenate(
            [g2[:, :3] * c, jnp.ones((BE, 1), jnp.float32),
             jnp.zeros((BE, SW - 4), jnp.float32)], axis=1)
        if lean:
            outs[0][...] = w2
        else:
            outs[0][...] = m
            outs[1][...] = w2

    out_specs = [pl.BlockSpec((BE, SW), lambda i: (i, 0))]
    out_shape = [jax.ShapeDtypeStruct((E, SW), jnp.float32)]
    if not lean:
        out_specs.insert(0, pl.BlockSpec((BE, D), lambda i: (i, 0)))
        out_shape.insert(0, jax.ShapeDtypeStruct((E, D), jnp.float32))
    return pl.pallas_call(
        body,
        grid=(E // BE,),
        in_specs=[
            pl.BlockSpec((BE, D), lambda i: (i, 0)),
            pl.BlockSpec((BE, SW), lambda i: (i, 0)),
            pl.BlockSpec((BE, ED), lambda i: (i, 0)),
            _const_spec((ED, D)),
            _const_spec((1, D)),
            _const_spec((1, D)),
            _const_spec((D, D)),
            _const_spec((1, D)),
            _const_spec((D, D)),
            _const_spec((1, D)),
            _const_spec((1, D)),
            _const_spec((1, 1)),
        ],
        out_specs=out_specs,
        out_shape=out_shape,
    )


def _posu(parts32_ref, pos_ref):
    s32 = jnp.sum(parts32_ref[...], axis=0)
    pd = jnp.concatenate([s32[:, :3], jnp.zeros((BN, 1), jnp.float32)], axis=1)
    denom = jnp.maximum(s32[:, 3:4], 1.0)
    return pos_ref[...] + pd / denom


@functools.cache
def _tc_node(N, D):
    def body(partsm_ref, parts32_ref, h_ref, pos_ref, wn1a_ref, wn1b_ref,
             bn1_ref, wn2_ref, bn2_ref, wa_ref, wb_ref,
             h_out, pos_out, a_out, b_out):
        p = partsm_ref[...]
        agg = p[0] + p[1]
        pos_out[...] = _posu(parts32_ref, pos_ref)
        h = h_ref[...]
        u1 = _silu(jnp.dot(h, wn1a_ref[...], preferred_element_type=jnp.float32)
                   + jnp.dot(agg, wn1b_ref[...], preferred_element_type=jnp.float32)
                   + bn1_ref[...])
        ho = h + jnp.dot(u1, wn2_ref[...], preferred_element_type=jnp.float32) + bn2_ref[...]
        h_out[...] = ho
        a_out[...] = jnp.dot(ho, wa_ref[...], preferred_element_type=jnp.float32)
        b_out[...] = jnp.dot(ho, wb_ref[...], preferred_element_type=jnp.float32)

    return pl.pallas_call(
        body,
        grid=(N // BN,),
        in_specs=[
            pl.BlockSpec((NC, BN, D), lambda i: (0, i, 0)),
            pl.BlockSpec((NW, BN, SW), lambda i: (0, i, 0)),
            pl.BlockSpec((BN, D), lambda i: (i, 0)),
            pl.BlockSpec((BN, 4), lambda i: (i, 0)),
            _const_spec((D, D)), _const_spec((D, D)), _const_spec((1, D)),
            _const_spec((D, D)), _const_spec((1, D)),
            _const_spec((D, D)), _const_spec((D, D)),
        ],
        out_specs=[
            pl.BlockSpec((BN, D), lambda i: (i, 0)),
            pl.BlockSpec((BN, 4), lambda i: (i, 0)),
            pl.BlockSpec((BN, D), lambda i: (i, 0)),
            pl.BlockSpec((BN, D), lambda i: (i, 0)),
        ],
        out_shape=[
            jax.ShapeDtypeStruct((N, D), jnp.float32),
            jax.ShapeDtypeStruct((N, 4), jnp.float32),
            jax.ShapeDtypeStruct((N, D), jnp.float32),
            jax.ShapeDtypeStruct((N, D), jnp.float32),
        ],
    )


@functools.cache
def _tc_pos(N):
    def body(parts32_ref, pos_ref, pos_out):
        pos_out[...] = _posu(parts32_ref, pos_ref)

    return pl.pallas_call(
        body,
        grid=(N // BN,),
        in_specs=[
            pl.BlockSpec((NW, BN, SW), lambda i: (0, i, 0)),
            pl.BlockSpec((BN, 4), lambda i: (i, 0)),
        ],
        out_specs=pl.BlockSpec((BN, 4), lambda i: (i, 0)),
        out_shape=jax.ShapeDtypeStruct((N, 4), jnp.float32),
    )


def kernel(x, pos, edge_index, edge_attr, We1, be1, We2, be2,
           Wc1, bc1, Wc2, bc2, Wn1, bn1, Wn2, bn2):
    N, D = x.shape
    E = edge_index.shape[1]
    ED = edge_attr.shape[1]
    L = We1.shape[0]
    src = edge_index[0]
    dst = edge_index[1]

    NP = -(-N // (NS * 8)) * (NS * 8)
    pos4 = jnp.pad(pos, ((0, 0), (0, 1)))
    z8 = jnp.zeros((N * SW,), jnp.float32)
    zrows = jnp.zeros((NP, D), jnp.float32)

    W1a = We1[:, :D, :]
    W1b = We1[:, D:2 * D, :]
    wd2 = We1[:, 2 * D:2 * D + 1, :]
    W1ea = We1[:, 2 * D + 1:, :]
    Wn1a = Wn1[:, :D, :]
    Wn1b = Wn1[:, D:, :]
    wc2row = jnp.transpose(Wc2, (0, 2, 1))

    h = x
    A, B = _tc_tables(N, D)(x, W1a[0], W1b[0])
    for l in range(L):
        posf = pos4.reshape(-1)
        s, g2f = _sc_gather(E, N, D)(A, B, dst, src, posf)
        g2 = g2f.reshape(E, SW)
        last = l == L - 1
        wargs = (edge_attr, W1ea[l], wd2[l], be1[l][None], We2[l], be2[l][None],
                 Wc1[l], bc1[l][None], wc2row[l], bc2[l][None])
        if not last:
            m, w2 = _tc_edge(E, D, ED, False)(s, g2, *wargs)
            partsm = _sc_scatter_wide(E, N, D)(m, dst, zrows)
            parts32 = _sc_scatter_narrow(E, N)(w2.reshape(-1), dst, z8)
            h, pos4, A, B = _tc_node(N, D)(
                partsm.reshape(NC, NP, D), parts32.reshape(NW, N, SW), h, pos4,
                Wn1a[l], Wn1b[l], bn1[l][None], Wn2[l], bn2[l][None],
                W1a[l + 1], W1b[l + 1])
        else:
            (w2,) = _tc_edge(E, D, ED, True)(s, g2, *wargs)
            parts32 = _sc_scatter_narrow(E, N)(w2.reshape(-1), dst, z8)
            pos4 = _tc_pos(N)(parts32.reshape(NW, N, SW), pos4)
    return pos4[:, :3]


# speedup vs baseline: 3.9421x; 1.7104x over previous
import functools

import jax
import jax.numpy as jnp
from jax import lax
from jax.experimental import pallas as pl
from jax.experimental.pallas import tpu as pltpu
from jax.experimental.pallas import tpu_sc as plsc

NC = 2
NS = 16
NW = NC * NS
LANES = 16
CH = 80
SW = 8
BE = 2560
BN = 1000


def _silu(v):
    return v * jax.nn.sigmoid(v)


def _mesh():
    return plsc.VectorSubcoreMesh(core_axis_name="c", subcore_axis_name="s")


def _iota16():
    return jnp.arange(LANES, dtype=jnp.int32)


@functools.cache
def _sc_gather(E, N, D):
    EPW = E // NW
    NB = 5
    GCH = NB * CH
    assert EPW % GCH == 0

    @functools.partial(
        pl.kernel,
        out_type=(
            jax.ShapeDtypeStruct((E, D), jnp.float32),
            jax.ShapeDtypeStruct((E * SW,), jnp.float32),
        ),
        mesh=_mesh(),
        compiler_params=pltpu.CompilerParams(needs_layout_passes=False),
        scratch_types=[
            pltpu.VMEM((GCH,), jnp.int32),
            pltpu.VMEM((GCH,), jnp.int32),
            pltpu.VMEM((NB, CH, D), jnp.float32),
            pltpu.VMEM((GCH * SW,), jnp.float32),
            pltpu.VMEM((4 * N,), jnp.float32),
        ] + [pltpu.SemaphoreType.DMA] * NB,
    )
    def k(ta, tb, dsti, srci, posf, s_out, g2_out,
          idx_d, idx_s, bufs, buf2, posv, *sems):
        wid = lax.axis_index("s") * NC + lax.axis_index("c")
        base = wid * EPW
        pltpu.sync_copy(posf, posv)

        def body(g, carry):
            off0 = base + g * GCH
            pltpu.sync_copy(dsti.at[pl.ds(off0, GCH)], idx_d)
            pltpu.sync_copy(srci.at[pl.ds(off0, GCH)], idx_s)
            tds = [pltpu.async_copy(ta.at[idx_d.at[pl.ds(b * CH, CH)]],
                                    bufs.at[b], sems[b]) for b in range(NB)]
            for j in range(GCH // LANES):
                dm = idx_d[pl.ds(j * LANES, LANES)]
                sm = idx_s[pl.ds(j * LANES, LANES)]
                li = (j * LANES + _iota16()) * SW
                d2 = jnp.zeros((LANES,), jnp.float32)
                for c in range(3):
                    pd = plsc.load_gather(posv, [dm * 4 + c])
                    ps = plsc.load_gather(posv, [sm * 4 + c])
                    rel = pd - ps
                    d2 = d2 + rel * rel
                    plsc.store_scatter(buf2, [li + c], rel)
                plsc.store_scatter(buf2, [li + 3], d2)
            tss = []
            for b in range(NB):
                tds[b].wait()
                tss.append(pltpu.async_copy(tb.at[idx_s.at[pl.ds(b * CH, CH)]],
                                            bufs.at[b], sems[b], add=True))
            outs = []
            for b in range(NB):
                tss[b].wait()
                outs.append(pltpu.async_copy(
                    bufs.at[b], s_out.at[pl.ds(off0 + b * CH, CH)], sems[b]))
            pltpu.sync_copy(buf2, g2_out.at[pl.ds(off0 * SW, GCH * SW)])
            for b in range(NB):
                outs[b].wait()
            return carry

        lax.fori_loop(0, EPW // GCH, body, 0)

    return k


@functools.cache
def _sc_scatter_wide(E, N, D):
    EPW = E // NW
    NP = -(-N // (NS * 8)) * (NS * 8)
    NPT = NP // NS
    NB = 5
    CHW = 40
    GCH = NB * CHW
    assert EPW % GCH == 0

    @functools.partial(
        pl.kernel,
        out_type=jax.ShapeDtypeStruct((NC * NP, D), jnp.float32),
        mesh=_mesh(),
        compiler_params=pltpu.CompilerParams(needs_layout_passes=False),
        scratch_types=[
            pltpu.VMEM((NB, CHW), jnp.int32),
            pltpu.VMEM((NB, CHW, D), jnp.float32),
            pltpu.VMEM_SHARED((NP, D), jnp.float32),
        ] + [pltpu.SemaphoreType.DMA] * NB,
    )
    def k(m, dsti, zrows, partsm, idxb, bufs, accm, *sems):
        cid = lax.axis_index("c")
        sid = lax.axis_index("s")
        wid = sid * NC + cid
        base = wid * EPW
        pltpu.sync_copy(zrows.at[pl.ds(sid * NPT, NPT)],
                        accm.at[pl.ds(sid * NPT, NPT)])
        plsc.subcore_barrier()

        def body(g, carry):
            off0 = base + g * GCH
            ixs, mms = [], []
            for b in range(NB):
                off = off0 + b * CHW
                ixs.append(pltpu.async_copy(dsti.at[pl.ds(off, CHW)],
                                            idxb.at[b], sems[b]))
                mms.append(pltpu.async_copy(m.at[pl.ds(off, CHW)],
                                            bufs.at[b], sems[b]))
            scs = []
            for b in range(NB):
                ixs[b].wait()
                mms[b].wait()
                scs.append(pltpu.async_copy(bufs.at[b], accm.at[idxb.at[b]],
                                            sems[b], add=True))
            for b in range(NB):
                scs[b].wait()
            return carry

        lax.fori_loop(0, EPW // GCH, body, 0)
        plsc.subcore_barrier()
        pltpu.sync_copy(accm.at[pl.ds(sid * NPT, NPT)],
                        partsm.at[pl.ds(cid * NP + sid * NPT, NPT)])

    return k


@functools.cache
def _sc_scatter_narrow(E, N):
    EPW = E // NW
    CHN = 400
    assert EPW % CHN == 0

    @functools.partial(
        pl.kernel,
        out_type=jax.ShapeDtypeStruct((NW * N * SW,), jnp.float32),
        mesh=_mesh(),
        compiler_params=pltpu.CompilerParams(needs_layout_passes=False),
        scratch_types=[
            pltpu.VMEM((CHN,), jnp.int32),
            pltpu.VMEM((CHN * SW,), jnp.float32),
            pltpu.VMEM((N * SW,), jnp.float32),
            pltpu.SemaphoreType.DMA,
        ],
    )
    def k(w2f, dsti, z8, parts32, idx, buf2, acc2, sem):
        wid = lax.axis_index("s") * NC + lax.axis_index("c")
        base = wid * EPW
        pltpu.sync_copy(z8, acc2)

        def body(i, carry):
            off = base + i * CHN
            cp1 = pltpu.async_copy(dsti.at[pl.ds(off, CHN)], idx, sem)
            cp2 = pltpu.async_copy(w2f.at[pl.ds(off * SW, CHN * SW)], buf2, sem)
            cp1.wait()
            cp2.wait()
            for j in range(CHN // LANES):
                dm = idx[pl.ds(j * LANES, LANES)]
                li = (j * LANES + _iota16()) * SW
                for c in range(4):
                    v = plsc.load_gather(buf2, [li + c])
                    plsc.addupdate_scatter(acc2, [dm * SW + c], v)
            return carry

        lax.fori_loop(0, EPW // CHN, body, 0)
        pltpu.sync_copy(acc2, parts32.at[pl.ds(wid * N * SW, N * SW)])

    return k


def _const_spec(shape):
    return pl.BlockSpec(shape, lambda i: (0,) * len(shape))


@functools.cache
def _tc_tables(N, D):
    def body(h_ref, wa_ref, wb_ref, a_ref, b_ref):
        h = h_ref[...]
        a_ref[...] = jnp.dot(h, wa_ref[...], preferred_element_type=jnp.float32)
        b_ref[...] = jnp.dot(h, wb_ref[...], preferred_element_type=jnp.float32)

    return pl.pallas_call(
        body,
        grid=(N // BN,),
        in_specs=[
            pl.BlockSpec((BN, D), lambda i: (i, 0)),
            _const_spec((D, D)),
            _const_spec((D, D)),
        ],
        out_specs=[pl.BlockSpec((BN, D), lambda i: (i, 0))] * 2,
        out_shape=[jax.ShapeDtypeStruct((N, D), jnp.float32)] * 2,
    )


@functools.cache
def _tc_edge(E, D, ED, lean):
    def body(s_ref, g2_ref, ea_ref, w1ea_ref, wd2_ref, be1_ref, we2_ref,
             be2_ref, wc1_ref, bc1_ref, wc2_ref, bc2_ref, *outs):
        s = s_ref[...]
        g2 = g2_ref[...]
        d2 = g2[:, 3:4]
        pre = (s + d2 * wd2_ref[...] + be1_ref[...]
               + jnp.dot(ea_ref[...], w1ea_ref[...], preferred_element_type=jnp.float32))
        m1 = _silu(pre)
        m = _silu(jnp.dot(m1, we2_ref[...], preferred_element_type=jnp.float32)
                  + be2_ref[...])
        t = _silu(jnp.dot(m, wc1_ref[...], preferred_element_type=jnp.float32)
                  + bc1_ref[...])
        c = jnp.sum(t * wc2_ref[...], axis=1, keepdims=True) + bc2_ref[...]
        w2 = jnp.concatenate(
            [g2[:, :3] * c, jnp.ones((BE, 1), jnp.float32),
             jnp.zeros((BE, SW - 4), jnp.float32)], axis=1)
        if lean:
            outs[0][...] = w2
        else:
            outs[0][...] = m
            outs[1][...] = w2

    out_specs = [pl.BlockSpec((BE, SW), lambda i: (i, 0))]
    out_shape = [jax.ShapeDtypeStruct((E, SW), jnp.float32)]
    if not lean:
        out_specs.insert(0, pl.BlockSpec((BE, D), lambda i: (i, 0)))
        out_shape.insert(0, jax.ShapeDtypeStruct((E, D), jnp.float32))
    return pl.pallas_call(
        body,
        grid=(E // BE,),
        compiler_params=pltpu.CompilerParams(
            dimension_semantics=("arbitrary",)),
        in_specs=[
            pl.BlockSpec((BE, D), lambda i: (i, 0)),
            pl.BlockSpec((BE, SW), lambda i: (i, 0)),
            pl.BlockSpec((BE, ED), lambda i: (i, 0)),
            _const_spec((ED, D)),
            _const_spec((1, D)),
            _const_spec((1, D)),
            _const_spec((D, D)),
            _const_spec((1, D)),
            _const_spec((D, D)),
            _const_spec((1, D)),
            _const_spec((1, D)),
            _const_spec((1, 1)),
        ],
        out_specs=out_specs,
        out_shape=out_shape,
    )


def _posu(parts32_ref, pos_ref):
    s32 = jnp.sum(parts32_ref[...], axis=0)
    pd = jnp.concatenate([s32[:, :3], jnp.zeros((BN, 1), jnp.float32)], axis=1)
    denom = jnp.maximum(s32[:, 3:4], 1.0)
    return pos_ref[...] + pd / denom


@functools.cache
def _tc_node(N, D):
    def body(partsm_ref, parts32_ref, h_ref, pos_ref, wn1a_ref, wn1b_ref,
             bn1_ref, wn2_ref, bn2_ref, wa_ref, wb_ref,
             h_out, pos_out, a_out, b_out):
        p = partsm_ref[...]
        agg = p[0] + p[1]
        pos_out[...] = _posu(parts32_ref, pos_ref)
        h = h_ref[...]
        u1 = _silu(jnp.dot(h, wn1a_ref[...], preferred_element_type=jnp.float32)
                   + jnp.dot(agg, wn1b_ref[...], preferred_element_type=jnp.float32)
                   + bn1_ref[...])
        ho = h + jnp.dot(u1, wn2_ref[...], preferred_element_type=jnp.float32) + bn2_ref[...]
        h_out[...] = ho
        a_out[...] = jnp.dot(ho, wa_ref[...], preferred_element_type=jnp.float32)
        b_out[...] = jnp.dot(ho, wb_ref[...], preferred_element_type=jnp.float32)

    return pl.pallas_call(
        body,
        grid=(N // BN,),
        in_specs=[
            pl.BlockSpec((NC, BN, D), lambda i: (0, i, 0)),
            pl.BlockSpec((NW, BN, SW), lambda i: (0, i, 0)),
            pl.BlockSpec((BN, D), lambda i: (i, 0)),
            pl.BlockSpec((BN, 4), lambda i: (i, 0)),
            _const_spec((D, D)), _const_spec((D, D)), _const_spec((1, D)),
            _const_spec((D, D)), _const_spec((1, D)),
            _const_spec((D, D)), _const_spec((D, D)),
        ],
        out_specs=[
            pl.BlockSpec((BN, D), lambda i: (i, 0)),
            pl.BlockSpec((BN, 4), lambda i: (i, 0)),
            pl.BlockSpec((BN, D), lambda i: (i, 0)),
            pl.BlockSpec((BN, D), lambda i: (i, 0)),
        ],
        out_shape=[
            jax.ShapeDtypeStruct((N, D), jnp.float32),
            jax.ShapeDtypeStruct((N, 4), jnp.float32),
            jax.ShapeDtypeStruct((N, D), jnp.float32),
            jax.ShapeDtypeStruct((N, D), jnp.float32),
        ],
    )


@functools.cache
def _tc_pos(N):
    def body(parts32_ref, pos_ref, pos_out):
        pos_out[...] = _posu(parts32_ref, pos_ref)

    return pl.pallas_call(
        body,
        grid=(N // BN,),
        in_specs=[
            pl.BlockSpec((NW, BN, SW), lambda i: (0, i, 0)),
            pl.BlockSpec((BN, 4), lambda i: (i, 0)),
        ],
        out_specs=pl.BlockSpec((BN, 4), lambda i: (i, 0)),
        out_shape=jax.ShapeDtypeStruct((N, 4), jnp.float32),
    )


def kernel(x, pos, edge_index, edge_attr, We1, be1, We2, be2,
           Wc1, bc1, Wc2, bc2, Wn1, bn1, Wn2, bn2):
    N, D = x.shape
    E = edge_index.shape[1]
    ED = edge_attr.shape[1]
    L = We1.shape[0]
    src = edge_index[0]
    dst = edge_index[1]

    NP = -(-N // (NS * 8)) * (NS * 8)
    pos4 = jnp.pad(pos, ((0, 0), (0, 1)))
    z8 = jnp.zeros((N * SW,), jnp.float32)
    zrows = jnp.zeros((NP, D), jnp.float32)

    W1a = We1[:, :D, :]
    W1b = We1[:, D:2 * D, :]
    wd2 = We1[:, 2 * D:2 * D + 1, :]
    W1ea = We1[:, 2 * D + 1:, :]
    Wn1a = Wn1[:, :D, :]
    Wn1b = Wn1[:, D:, :]
    wc2row = jnp.transpose(Wc2, (0, 2, 1))

    h = x
    A, B = _tc_tables(N, D)(x, W1a[0], W1b[0])
    for l in range(L):
        posf = pos4.reshape(-1)
        s, g2f = _sc_gather(E, N, D)(A, B, dst, src, posf)
        g2 = g2f.reshape(E, SW)
        last = l == L - 1
        wargs = (edge_attr, W1ea[l], wd2[l], be1[l][None], We2[l], be2[l][None],
                 Wc1[l], bc1[l][None], wc2row[l], bc2[l][None])
        if not last:
            m, w2 = _tc_edge(E, D, ED, False)(s, g2, *wargs)
            partsm = _sc_scatter_wide(E, N, D)(m, dst, zrows)
            parts32 = _sc_scatter_narrow(E, N)(w2.reshape(-1), dst, z8)
            h, pos4, A, B = _tc_node(N, D)(
                partsm.reshape(NC, NP, D), parts32.reshape(NW, N, SW), h, pos4,
                Wn1a[l], Wn1b[l], bn1[l][None], Wn2[l], bn2[l][None],
                W1a[l + 1], W1b[l + 1])
        else:
            (w2,) = _tc_edge(E, D, ED, True)(s, g2, *wargs)
            parts32 = _sc_scatter_narrow(E, N)(w2.reshape(-1), dst, z8)
            pos4 = _tc_pos(N)(parts32.reshape(NW, N, SW), pos4)
    return pos4[:, :3]
